# baseline (speedup 1.0000x reference)
import jax
import jax.numpy as jnp
from jax.experimental import pallas as pl
from jax.experimental.pallas import tpu as pltpu

R = 512
NBUF = 2


def _body(idx_ref, h_ref, cv_ref, o_ref, in0, in1, out0, out1,
          is0, is1, os0, os1):
    B, S, E = h_ref.shape
    cpb = S // R
    nch = B * cpb
    ins = (in0, in1)
    outs = (out0, out1)
    isems = (is0, is1)
    osems = (os0, os1)

    def start_in(j, i):
        b = i // cpb
        r0 = (i % cpb) * R
        pltpu.make_async_copy(
            h_ref.at[b, pl.ds(r0, R)], ins[j], isems[j]
        ).start()

    def wait_in(j):
        pltpu.make_async_copy(h_ref.at[0, pl.ds(0, R)], ins[j], isems[j]).wait()

    def start_out(j, i):
        b = i // cpb
        r0 = (i % cpb) * R
        pltpu.make_async_copy(
            outs[j], o_ref.at[b, pl.ds(r0, R)], osems[j]
        ).start()

    def wait_out(j):
        pltpu.make_async_copy(outs[j], o_ref.at[0, pl.ds(0, R)], osems[j]).wait()

    start_in(0, 0)
    start_in(1, 1)

    @pl.loop(0, nch, step=NBUF)
    def _(i):
        for j in range(NBUF):
            ii = i + j
            b = ii // cpb
            wait_in(j)

            @pl.when(ii >= NBUF)
            def _():
                wait_out(j)

            adj = cv_ref[pl.ds(idx_ref[b], 1)]
            outs[j][...] = ins[j][...] + adj[0]

            start_out(j, ii)

            @pl.when(ii + NBUF < nch)
            def _():
                start_in(j, ii + NBUF)

    wait_out(0)
    wait_out(1)


def kernel(hidden_states, affective_state_indices, control_vectors):
    B, S, E = hidden_states.shape
    n = control_vectors.shape[0]
    idx = jnp.clip(affective_state_indices.astype(jnp.int32), 0, n - 1)
    cv3 = control_vectors.reshape(n, 1, E)

    return pl.pallas_call(
        _body,
        grid_spec=pltpu.PrefetchScalarGridSpec(
            num_scalar_prefetch=1,
            grid=(1,),
            in_specs=[
                pl.BlockSpec(memory_space=pltpu.MemorySpace.HBM),
                pl.BlockSpec((n, 1, E), lambda g, idx_ref: (0, 0, 0)),
            ],
            out_specs=pl.BlockSpec(memory_space=pltpu.MemorySpace.HBM),
            scratch_shapes=[
                pltpu.VMEM((R, E), jnp.float32),
                pltpu.VMEM((R, E), jnp.float32),
                pltpu.VMEM((R, E), jnp.float32),
                pltpu.VMEM((R, E), jnp.float32),
                pltpu.SemaphoreType.DMA,
                pltpu.SemaphoreType.DMA,
                pltpu.SemaphoreType.DMA,
                pltpu.SemaphoreType.DMA,
            ],
        ),
        out_shape=jax.ShapeDtypeStruct((B, S, E), hidden_states.dtype),
    )(idx, hidden_states, cv3)


# speedup vs baseline: 1.0581x; 1.0581x over previous
import jax
import jax.numpy as jnp
from jax.experimental import pallas as pl
from jax.experimental.pallas import tpu as pltpu

R = 256
NBUF = 4


def _body(idx_ref, h_ref, cv_ref, o_ref, in0, in1, in2, in3,
          out0, out1, out2, out3, is0, is1, is2, is3, os0, os1, os2, os3):
    B, S, E = h_ref.shape
    cpb = S // R
    nch = B * cpb
    ins = (in0, in1, in2, in3)
    outs = (out0, out1, out2, out3)
    isems = (is0, is1, is2, is3)
    osems = (os0, os1, os2, os3)

    def start_in(j, i):
        b = i // cpb
        r0 = (i % cpb) * R
        pltpu.make_async_copy(
            h_ref.at[b, pl.ds(r0, R)], ins[j], isems[j]
        ).start()

    def wait_in(j):
        pltpu.make_async_copy(h_ref.at[0, pl.ds(0, R)], ins[j], isems[j]).wait()

    def start_out(j, i):
        b = i // cpb
        r0 = (i % cpb) * R
        pltpu.make_async_copy(
            outs[j], o_ref.at[b, pl.ds(r0, R)], osems[j]
        ).start()

    def wait_out(j):
        pltpu.make_async_copy(outs[j], o_ref.at[0, pl.ds(0, R)], osems[j]).wait()

    for j in range(NBUF):
        start_in(j, j)

    @pl.loop(0, nch, step=NBUF)
    def _(i):
        for j in range(NBUF):
            ii = i + j
            b = ii // cpb
            wait_in(j)

            @pl.when(ii >= NBUF)
            def _():
                wait_out(j)

            adj = cv_ref[pl.ds(idx_ref[b], 1)]
            outs[j][...] = ins[j][...] + adj[0]

            start_out(j, ii)

            @pl.when(ii + NBUF < nch)
            def _():
                start_in(j, ii + NBUF)

    for j in range(NBUF):
        wait_out(j)


def kernel(hidden_states, affective_state_indices, control_vectors):
    B, S, E = hidden_states.shape
    n = control_vectors.shape[0]
    idx = jnp.clip(affective_state_indices.astype(jnp.int32), 0, n - 1)
    cv3 = control_vectors.reshape(n, 1, E)

    return pl.pallas_call(
        _body,
        grid_spec=pltpu.PrefetchScalarGridSpec(
            num_scalar_prefetch=1,
            grid=(1,),
            in_specs=[
                pl.BlockSpec(memory_space=pltpu.MemorySpace.HBM),
                pl.BlockSpec((n, 1, E), lambda g, idx_ref: (0, 0, 0)),
            ],
            out_specs=pl.BlockSpec(memory_space=pltpu.MemorySpace.HBM),
            scratch_shapes=[
                *([pltpu.VMEM((R, E), jnp.float32)] * 8),
                *([pltpu.SemaphoreType.DMA] * 8),
            ],
        ),
        out_shape=jax.ShapeDtypeStruct((B, S, E), hidden_states.dtype),
    )(idx, hidden_states, cv3)


# speedup vs baseline: 1.0854x; 1.0259x over previous
import jax
import jax.numpy as jnp
from jax.experimental import pallas as pl
from jax.experimental.pallas import tpu as pltpu


def _body(idx_ref, h_ref, cv_ref, o_ref):
    b = pl.program_id(0)
    adj = cv_ref[pl.ds(idx_ref[b], 1)]
    o_ref[...] = h_ref[...] + adj[0]


def kernel(hidden_states, affective_state_indices, control_vectors):
    B, S, E = hidden_states.shape
    n = control_vectors.shape[0]
    idx = jnp.clip(affective_state_indices.astype(jnp.int32), 0, n - 1)
    cv3 = control_vectors.reshape(n, 1, E)
    BS = 2048
    grid = (B, S // BS)

    def h_map(b, s, idx_ref):
        return (b, s, 0)

    return pl.pallas_call(
        _body,
        grid_spec=pltpu.PrefetchScalarGridSpec(
            num_scalar_prefetch=1,
            grid=grid,
            in_specs=[
                pl.BlockSpec((1, BS, E), h_map),
                pl.BlockSpec((n, 1, E), lambda b, s, idx_ref: (0, 0, 0)),
            ],
            out_specs=pl.BlockSpec((1, BS, E), h_map),
        ),
        out_shape=jax.ShapeDtypeStruct((B, S, E), hidden_states.dtype),
    )(idx, hidden_states, cv3)


# speedup vs baseline: 1.0931x; 1.0071x over previous
import jax
import jax.numpy as jnp
from jax.experimental import pallas as pl
from jax.experimental.pallas import tpu as pltpu


def _body(idx_ref, h_ref, cv_ref, o_ref):
    o_ref[...] = h_ref[...] + cv_ref[0]


def kernel(hidden_states, affective_state_indices, control_vectors):
    B, S, E = hidden_states.shape
    n = control_vectors.shape[0]
    idx = affective_state_indices.astype(jnp.int32)
    cv3 = control_vectors.reshape(n, 1, E)
    BS = 2048
    grid = (B, S // BS)

    def h_map(b, s, idx_ref):
        return (b, s, 0)

    def cv_map(b, s, idx_ref):
        return (jnp.clip(idx_ref[b], 0, n - 1), 0, 0)

    return pl.pallas_call(
        _body,
        grid_spec=pltpu.PrefetchScalarGridSpec(
            num_scalar_prefetch=1,
            grid=grid,
            in_specs=[
                pl.BlockSpec((1, BS, E), h_map),
                pl.BlockSpec((1, 1, E), cv_map),
            ],
            out_specs=pl.BlockSpec((1, BS, E), h_map),
        ),
        out_shape=jax.ShapeDtypeStruct((B, S, E), hidden_states.dtype),
    )(idx, hidden_states, cv3)
